# baseline (speedup 1.0000x reference)
import numpy as np
import jax
import jax.numpy as jnp
from jax import lax
from jax.experimental import pallas as pl
from jax.experimental.pallas import tpu as pltpu
from jax.experimental.pallas import tpu_sc as plsc

NUM_STEPS = 100
K = 20
N = 128
L = 1024

_NW = 32
_ROWS_PER_W = N // _NW
_C = 128


def _np_threefry2x32(k1, k2, x1, x2):
    def rotl(x, r):
        return (x << np.uint32(r)) | (x >> np.uint32(32 - r))

    ks = [np.uint32(k1), np.uint32(k2), np.uint32(k1 ^ k2 ^ 0x1BD11BDA)]
    x = [x1.astype(np.uint32).copy(), x2.astype(np.uint32).copy()]
    rotations = [[13, 15, 26, 6], [17, 29, 16, 24]]
    x[0] += ks[0]
    x[1] += ks[1]
    for i in range(5):
        for r in rotations[i % 2]:
            x[0] += x[1]
            x[1] = rotl(x[1], r)
            x[1] ^= x[0]
        x[0] += ks[(i + 1) % 3]
        x[1] += ks[(i + 2) % 3] + np.uint32(i + 1)
    return x


def _gumbel_exp_table():
    n = N * L * K
    lo = np.arange(n, dtype=np.uint32)
    hi = np.zeros(n, dtype=np.uint32)
    o1, o2 = _np_threefry2x32(0, 1, hi, lo)
    bits = o1 ^ o2
    f = ((bits >> np.uint32(9)) | np.uint32(0x3F800000)).view(np.float32)
    f = f - np.float32(1.0)
    tiny = np.float32(np.finfo(np.float32).tiny)
    u = np.maximum(tiny, f * (np.float32(1.0) - tiny) + tiny)
    t = (-1.0 / np.log(u.astype(np.float64))).astype(np.float32)
    return np.ascontiguousarray(t.reshape(N, L, K).transpose(0, 2, 1))


_T_TABLE = _gumbel_exp_table()


def _sc_body(c0_hbm, x_hbm, m_hbm, p_hbm, t_hbm, post_hbm, out_hbm,
             c0_v0, c0_v1, t_v0, t_v1, x_v, m_v, p_v,
             post_v0, post_v1, out_v0, out_v1,
             s_c00, s_c01, s_t0, s_t1, s_p0, s_p1, s_o0, s_o1):
    cid = lax.axis_index("c")
    sid = lax.axis_index("s")
    wid = sid * 2 + cid

    c0_v = (c0_v0, c0_v1)
    t_v = (t_v0, t_v1)
    post_v = (post_v0, post_v1)
    out_v = (out_v0, out_v1)
    s_c0 = (s_c00, s_c01)
    s_t = (s_t0, s_t1)
    s_post = (s_p0, s_p1)
    s_out = (s_o0, s_o1)

    lane = jax.lax.broadcasted_iota(jnp.int32, (16,), 0)
    nch = L // _C

    def in_cps(row, lo, par):
        return (
            pltpu.make_async_copy(c0_hbm.at[row, pl.ds(lo, _C)], c0_v[par], s_c0[par]),
            pltpu.make_async_copy(t_hbm.at[row, :, pl.ds(lo, _C)], t_v[par], s_t[par]),
        )

    def out_cps(row, lo, par):
        return (
            pltpu.make_async_copy(post_v[par], post_hbm.at[row, pl.ds(lo, _C)], s_post[par]),
            pltpu.make_async_copy(out_v[par], out_hbm.at[row, pl.ds(lo, _C)], s_out[par]),
        )

    for r in range(_ROWS_PER_W):
      row = wid * _ROWS_PER_W + r
      pltpu.sync_copy(p_hbm.at[row], p_v)
      pltpu.sync_copy(x_hbm.at[row], x_v)
      pltpu.sync_copy(m_hbm.at[row], m_v)
      a_lo = p_v[pl.ds(0, 16)]
      alpha = p_v[pl.ds(16, 16)]
      albar = p_v[pl.ds(32, 16)]
      b_lo = p_v[pl.ds(48, 16)]
      b20 = p_v[pl.ds(64, 16)]
      a_hi = p_v[pl.ds(80, 16)]

      for cp in in_cps(row, 0, 0):
          cp.start()

      def chunk2(j, _):
        for par in (0, 1):
          h = 2 * j + par
          lo = h * _C

          @pl.when(h + 1 < nch)
          def _():
              for cp in in_cps(row, lo + _C, 1 - par):
                  cp.start()

          for cp in in_cps(row, lo, par):
              cp.wait()

          @pl.when(h >= 2)
          def _():
              for cp in out_cps(row, lo - 2 * _C, par):
                  cp.wait()

          def group(i, _):
            base = i * 16
            ri = base + lane
            xv = x_v[pl.ds(lo + base, 16)]
            mv = m_v[pl.ds(lo + base, 16)]
            genm = mv != 0
            s = plsc.load_gather(c0_v[par], [ri, jnp.zeros((16,), jnp.int32)])
            for k in range(1, K):
                kk = jnp.full((16,), k, jnp.int32)
                s = s + plsc.load_gather(c0_v[par], [ri, kk])
            c0x = plsc.load_gather(c0_v[par], [ri, xv])
            like_x = albar * c0x + b_lo
            ssum = a_lo * (albar * s + b20) + alpha * like_x
            inv = 1.0 / (ssum + 1e-8)
            zero = jnp.zeros((16,), jnp.float32)
            ainv = a_lo * inv
            Am = jnp.where(genm, ainv * albar, zero)
            Bm = jnp.where(genm, ainv * b_lo, zero)
            fix = jnp.where(genm, (a_hi * inv) * like_x, jnp.ones((16,), jnp.float32))
            best_s = zero
            best_k = jnp.zeros((16,), jnp.int32)
            for k in range(K):
                kk = jnp.full((16,), k, jnp.int32)
                c0k = plsc.load_gather(c0_v[par], [ri, kk])
                postk = Am * c0k + Bm
                plsc.store_scatter(post_v[par], [ri, kk], postk)
                tk = t_v[par][k, pl.ds(base, 16)]
                score = (postk + 1e-8) * tk
                gt = score > best_s
                best_s = jnp.where(gt, score, best_s)
                best_k = jnp.where(gt, kk, best_k)
            plsc.store_scatter(post_v[par], [ri, xv], fix)
            tx = plsc.load_gather(t_v[par], [xv, ri])
            score_x = (fix + 1e-8) * tx
            gtx = score_x > best_s
            best_k = jnp.where(gtx, xv, best_k)
            out_v[par][pl.ds(base, 16)] = best_k
            return 0

          lax.fori_loop(0, _C // 16, group, 0)

          for cp in out_cps(row, lo, par):
              cp.start()
        return 0

      lax.fori_loop(0, nch // 2, chunk2, 0)

      for h in (nch - 2, nch - 1):
          for cp in out_cps(row, h * _C, h % 2):
              cp.wait()


def kernel(x_t, c0_pred, generation_mask, t, alphas, alpha_bars):
    a = alphas[t]
    ab = alpha_bars[jnp.clip(t - 1, 0, None)]
    a_lo = (1.0 - a) / K
    b_lo = (1.0 - ab) / K
    params = jnp.stack([a_lo, a, ab, b_lo, K * b_lo, a + a_lo], axis=1)
    params = jnp.broadcast_to(params[:, :, None], (N, 6, 16))
    params = params.astype(jnp.float32).reshape(N, 96)
    mask_i = generation_mask.astype(jnp.int32)
    tbl = jnp.asarray(_T_TABLE)

    mesh = plsc.VectorSubcoreMesh(core_axis_name="c", subcore_axis_name="s")
    sc = pl.kernel(
        _sc_body,
        mesh=mesh,
        compiler_params=pltpu.CompilerParams(needs_layout_passes=False),
        out_type=[
            jax.ShapeDtypeStruct((N, L, K), jnp.float32),
            jax.ShapeDtypeStruct((N, L), jnp.int32),
        ],
        scratch_types=[
            pltpu.VMEM((_C, K), jnp.float32),
            pltpu.VMEM((_C, K), jnp.float32),
            pltpu.VMEM((K, _C), jnp.float32),
            pltpu.VMEM((K, _C), jnp.float32),
            pltpu.VMEM((L,), jnp.int32),
            pltpu.VMEM((L,), jnp.int32),
            pltpu.VMEM((96,), jnp.float32),
            pltpu.VMEM((_C, K), jnp.float32),
            pltpu.VMEM((_C, K), jnp.float32),
            pltpu.VMEM((_C,), jnp.int32),
            pltpu.VMEM((_C,), jnp.int32),
        ] + [pltpu.SemaphoreType.DMA] * 8,
    )
    post, x_tm1 = sc(c0_pred, x_t, mask_i, params, tbl)
    return post, x_tm1


# speedup vs baseline: 2.0861x; 2.0861x over previous
import numpy as np
import jax
import jax.numpy as jnp
from jax import lax
from jax.experimental import pallas as pl
from jax.experimental.pallas import tpu as pltpu
from jax.experimental.pallas import tpu_sc as plsc

NUM_STEPS = 100
K = 20
N = 128
L = 1024

_NW = 32
_ROWS_PER_W = N // _NW
_C = 512


def _np_threefry2x32(k1, k2, x1, x2):
    def rotl(x, r):
        return (x << np.uint32(r)) | (x >> np.uint32(32 - r))

    ks = [np.uint32(k1), np.uint32(k2), np.uint32(k1 ^ k2 ^ 0x1BD11BDA)]
    x = [x1.astype(np.uint32).copy(), x2.astype(np.uint32).copy()]
    rotations = [[13, 15, 26, 6], [17, 29, 16, 24]]
    x[0] += ks[0]
    x[1] += ks[1]
    for i in range(5):
        for r in rotations[i % 2]:
            x[0] += x[1]
            x[1] = rotl(x[1], r)
            x[1] ^= x[0]
        x[0] += ks[(i + 1) % 3]
        x[1] += ks[(i + 2) % 3] + np.uint32(i + 1)
    return x


def _gumbel_exp_table():
    n = N * L * K
    lo = np.arange(n, dtype=np.uint32)
    hi = np.zeros(n, dtype=np.uint32)
    o1, o2 = _np_threefry2x32(0, 1, hi, lo)
    bits = o1 ^ o2
    f = ((bits >> np.uint32(9)) | np.uint32(0x3F800000)).view(np.float32)
    f = f - np.float32(1.0)
    tiny = np.float32(np.finfo(np.float32).tiny)
    u = np.maximum(tiny, f * (np.float32(1.0) - tiny) + tiny)
    t = (-1.0 / np.log(u.astype(np.float64))).astype(np.float32)
    return np.ascontiguousarray(t.reshape(N, L, K).transpose(0, 2, 1))


_T_TABLE = _gumbel_exp_table()


def _sc_body(c0_hbm, x_hbm, m_hbm, p_hbm, t_hbm, post_hbm, out_hbm,
             c0_v0, c0_v1, t_v0, t_v1, x_v, m_v, p_v,
             post_v0, post_v1, out_v0, out_v1,
             s_c00, s_c01, s_t0, s_t1, s_p0, s_p1, s_o0, s_o1):
    cid = lax.axis_index("c")
    sid = lax.axis_index("s")
    wid = sid * 2 + cid

    c0_v = (c0_v0, c0_v1)
    t_v = (t_v0, t_v1)
    post_v = (post_v0, post_v1)
    out_v = (out_v0, out_v1)
    s_c0 = (s_c00, s_c01)
    s_t = (s_t0, s_t1)
    s_post = (s_p0, s_p1)
    s_out = (s_o0, s_o1)

    lane = jax.lax.broadcasted_iota(jnp.int32, (16,), 0)
    lane20 = lane * K
    nch = L // _C
    ntot = _ROWS_PER_W * nch

    def in_cps(g, par):
        row, lo = wid * _ROWS_PER_W + g // nch, (g % nch) * _C
        return (
            pltpu.make_async_copy(c0_hbm.at[row, pl.ds(lo * K, _C * K)], c0_v[par], s_c0[par]),
            pltpu.make_async_copy(t_hbm.at[row, :, pl.ds(lo, _C)], t_v[par], s_t[par]),
        )

    def out_cps(g, par):
        row, lo = wid * _ROWS_PER_W + g // nch, (g % nch) * _C
        return (
            pltpu.make_async_copy(post_v[par], post_hbm.at[row, pl.ds(lo * K, _C * K)], s_post[par]),
            pltpu.make_async_copy(out_v[par], out_hbm.at[row, pl.ds(lo, _C)], s_out[par]),
        )

    for cp in in_cps(0, 0):
        cp.start()

    for g in range(ntot):
      par = g % 2
      row, lo = g // nch, (g % nch) * _C

      if g + 1 < ntot:
          for cp in in_cps(g + 1, 1 - par):
              cp.start()

      if g % nch == 0:
          pltpu.sync_copy(p_hbm.at[wid * _ROWS_PER_W + row], p_v)
          pltpu.sync_copy(x_hbm.at[wid * _ROWS_PER_W + row], x_v)
          pltpu.sync_copy(m_hbm.at[wid * _ROWS_PER_W + row], m_v)

      a_lo = p_v[pl.ds(0, 16)]
      albar = p_v[pl.ds(32, 16)]
      b_lo = p_v[pl.ds(48, 16)]
      a_hi = p_v[pl.ds(80, 16)]

      for cp in in_cps(g, par):
          cp.wait()
      if g >= 2:
          for cp in out_cps(g - 2, par):
              cp.wait()

      def group(i, _):
          base = i * 16
          fbase = base * K + lane20
          xv = x_v[pl.ds(lo + base, 16)]
          mv = m_v[pl.ds(lo + base, 16)]
          genm_f = jnp.where(mv != 0, jnp.ones((16,), jnp.float32),
                             jnp.zeros((16,), jnp.float32))
          notm = mv == 0
          th = []
          ssum = jnp.zeros((16,), jnp.float32)
          for k in range(K):
              kk = jnp.full((16,), k, jnp.int32)
              c0k = plsc.load_gather(c0_v[par], [fbase + kk])
              hot = xv == kk
              prior = jnp.where(hot, a_hi, a_lo)
              thk = prior * (albar * c0k + b_lo)
              th.append(thk)
              ssum = ssum + thk
          minv = genm_f * (1.0 / (ssum + 1e-8))
          best_s0 = jnp.zeros((16,), jnp.float32)
          best_k0 = jnp.zeros((16,), jnp.int32)
          best_s1 = jnp.zeros((16,), jnp.float32)
          best_k1 = jnp.zeros((16,), jnp.int32)
          for k in range(K):
              kk = jnp.full((16,), k, jnp.int32)
              postk = th[k] * minv
              plsc.store_scatter(post_v[par], [fbase + kk], postk)
              tk = t_v[par][k, pl.ds(base, 16)]
              score = (postk + 1e-8) * tk
              if k % 2 == 0:
                  gt = score > best_s0
                  best_s0 = jnp.where(gt, score, best_s0)
                  best_k0 = jnp.where(gt, kk, best_k0)
              else:
                  gt = score > best_s1
                  best_s1 = jnp.where(gt, score, best_s1)
                  best_k1 = jnp.where(gt, kk, best_k1)
          lt = best_s1 > best_s0
          best_s = jnp.where(lt, best_s1, best_s0)
          best_k = jnp.where(lt, best_k1, best_k0)
          one = jnp.ones((16,), jnp.float32)
          plsc.store_scatter(post_v[par], [fbase + xv], one, mask=notm)
          tx = plsc.load_gather(t_v[par], [xv, base + lane])
          score_x = (1.0 + 1e-8) * tx
          gtx = notm & (score_x > best_s)
          best_k = jnp.where(gtx, xv, best_k)
          out_v[par][pl.ds(base, 16)] = best_k
          return 0

      lax.fori_loop(0, _C // 16, group, 0)

      for cp in out_cps(g, par):
          cp.start()

    for g in (ntot - 2, ntot - 1):
        for cp in out_cps(g, g % 2):
            cp.wait()


def kernel(x_t, c0_pred, generation_mask, t, alphas, alpha_bars):
    a = alphas[t]
    ab = alpha_bars[jnp.clip(t - 1, 0, None)]
    a_lo = (1.0 - a) / K
    b_lo = (1.0 - ab) / K
    params = jnp.stack([a_lo, a, ab, b_lo, K * b_lo, a + a_lo], axis=1)
    params = jnp.broadcast_to(params[:, :, None], (N, 6, 16))
    params = params.astype(jnp.float32).reshape(N, 96)
    mask_i = generation_mask.astype(jnp.int32)
    tbl = jnp.asarray(_T_TABLE)

    mesh = plsc.VectorSubcoreMesh(core_axis_name="c", subcore_axis_name="s")
    sc = pl.kernel(
        _sc_body,
        mesh=mesh,
        compiler_params=pltpu.CompilerParams(needs_layout_passes=False),
        out_type=[
            jax.ShapeDtypeStruct((N, L * K), jnp.float32),
            jax.ShapeDtypeStruct((N, L), jnp.int32),
        ],
        scratch_types=[
            pltpu.VMEM((_C * K,), jnp.float32),
            pltpu.VMEM((_C * K,), jnp.float32),
            pltpu.VMEM((K, _C), jnp.float32),
            pltpu.VMEM((K, _C), jnp.float32),
            pltpu.VMEM((L,), jnp.int32),
            pltpu.VMEM((L,), jnp.int32),
            pltpu.VMEM((96,), jnp.float32),
            pltpu.VMEM((_C * K,), jnp.float32),
            pltpu.VMEM((_C * K,), jnp.float32),
            pltpu.VMEM((_C,), jnp.int32),
            pltpu.VMEM((_C,), jnp.int32),
        ] + [pltpu.SemaphoreType.DMA] * 8,
    )
    post_f, x_tm1 = sc(c0_pred.reshape(N, L * K), x_t, mask_i, params, tbl)
    return post_f.reshape(N, L, K), x_tm1
